# baseline (speedup 1.0000x reference)
import functools

import jax
import jax.numpy as jnp
from jax import lax
from jax.experimental import pallas as pl
from jax.experimental.pallas import tpu as pltpu
from jax.experimental.pallas import tpu_sc as plsc

P = 524288
NC = 2
NS = 16
NW = NC * NS
PPW = P // NW
CHUNK = 128
NCHUNKS = PPW // CHUNK
LANES = 16
NG = CHUNK // LANES

VOLS = ((64, 8, 1, 0, 0), (128, 4, 2, 1, 8))
CTOT = 14
RPL = 64
N2 = 256
V2ROWS = N2 * N2 * (N2 // 8)

V0 = 64 * 64 * 64
V1 = 128 * 128 * 128
K0 = 1024
K1 = 2048
KG = 2048


def _fmt_body(v0, v1, gr, t0f, t1f, gxf, gyf, gzf,
              inb0, inb1, outb, ing, og, sem):
    wid = lax.axis_index("s") * NC + lax.axis_index("c")
    lane = lax.iota(jnp.int32, LANES)

    def do_vol(vin, tout, inb, c, k, nchunks, wvox, sh):
        cvec = lane & (c - 1)
        vbase = lane >> sh
        step = 16 // c

        def chunk(ci, carry):
            base = wid * wvox + ci * k
            handles = [pltpu.async_copy(vin.at[ch, pl.ds(base, k)],
                                        inb.at[ch], sem) for ch in range(c)]
            for h in handles:
                h.wait()

            def vloop(m, c2):
                for u in range(8):
                    mm = m * 8 + u
                    vox = vbase + mm * step
                    outb[pl.ds(mm * LANES, LANES)] = plsc.load_gather(
                        inb, [cvec, vox])
                return c2

            lax.fori_loop(0, k * c // (LANES * 8), vloop, 0)
            pltpu.sync_copy(outb.at[pl.ds(0, k * c)],
                            tout.at[pl.ds(base * c, k * c)])
            return carry

        lax.fori_loop(0, nchunks, chunk, 0)

    do_vol(v0, t0f, inb0, 8, K0, V0 // NW // K0, V0 // NW, 3)
    do_vol(v1, t1f, inb1, 4, K1, V1 // NW // K1, V1 // NW, 2)

    gouts = (gxf, gyf, gzf)
    lane3 = lane * 3

    def gchunk(ci, carry):
        base = wid * PPW + ci * KG
        pltpu.sync_copy(gr.at[pl.ds(base * 3, KG * 3)], ing)
        for a in range(3):

            def vloop(m, c2):
                for u in range(8):
                    mm = m * 8 + u
                    og[pl.ds(mm * LANES, LANES)] = plsc.load_gather(
                        ing, [lane3 + (a + mm * 48)])
                return c2

            lax.fori_loop(0, KG // (LANES * 8), vloop, 0)
            pltpu.sync_copy(og, gouts[a].at[pl.ds(base, KG)])
        return carry

    lax.fori_loop(0, PPW // KG, gchunk, 0)


def _slot(refs, base):
    return dict(
        w0=refs[base + 0], w1=refs[base + 1], w2=refs[base + 2],
        i01=(refs[base + 3:base + 11], refs[base + 11:base + 19]),
        i2=refs[base + 19:base + 35],
        o01=(refs[base + 35:base + 37], refs[base + 37:base + 39]),
        o2=(refs[base + 39], refs[base + 40]),
        b01=(refs[base + 41:base + 49], refs[base + 49:base + 57]),
        b2=refs[base + 57:base + 73],
        sem=refs[base + 73],
    )


def _sc_body(t0, t1, t2, gx, gy, gz, out, *refs):
    cx, cy, cz, ob = refs[:4]
    slots = (_slot(refs, 4), _slot(refs, 4 + 74))
    wid = lax.axis_index("s") * NC + lax.axis_index("c")
    wbase = wid * PPW
    tabs = (t0, t1)
    lane = lax.iota(jnp.int32, LANES)

    def build(sl, pbase):
        pltpu.sync_copy(gx.at[pl.ds(pbase, CHUNK)], cx)
        pltpu.sync_copy(gy.at[pl.ds(pbase, CHUNK)], cy)
        pltpu.sync_copy(gz.at[pl.ds(pbase, CHUNK)], cz)
        wrefs = (sl["w0"], sl["w1"])

        def bloop(g, c2):
            s = g * LANES
            gxv = cx[pl.ds(s, LANES)]
            gyv = cy[pl.ds(s, LANES)]
            gzv = cz[pl.ds(s, LANES)]
            for v, (n, c, grp, sh, _cb) in enumerate(VOLS):
                scale = jnp.float32(0.5 * (n - 1))
                hi = jnp.float32(n - 1)
                tx = jnp.minimum(jnp.maximum((gxv + 1.0) * scale, 0.0), hi)
                ty = jnp.minimum(jnp.maximum((gyv + 1.0) * scale, 0.0), hi)
                tz = jnp.minimum(jnp.maximum((gzv + 1.0) * scale, 0.0), hi)
                x0 = tx.astype(jnp.int32)
                y0 = ty.astype(jnp.int32)
                z0 = tz.astype(jnp.int32)
                wrefs[v][0, pl.ds(s, LANES)] = tx - x0.astype(jnp.float32)
                wrefs[v][1, pl.ds(s, LANES)] = ty - y0.astype(jnp.float32)
                wrefs[v][2, pl.ds(s, LANES)] = tz - z0.astype(jnp.float32)
                x1 = jnp.minimum(x0 + 1, n - 1)
                y1 = jnp.minimum(y0 + 1, n - 1)
                z1 = jnp.minimum(z0 + 1, n - 1)
                xa = lax.shift_right_logical(x0, sh)
                xb = lax.shift_right_logical(x1, sh)
                sl["o01"][v][0][pl.ds(s, LANES)] = (x0 & (grp - 1)) * c
                sl["o01"][v][1][pl.ds(s, LANES)] = (x1 & (grp - 1)) * c
                l00 = (z0 * n + y0) * RPL
                l01 = (z0 * n + y1) * RPL
                l10 = (z1 * n + y0) * RPL
                l11 = (z1 * n + y1) * RPL
                iref = sl["i01"][v]
                iref[0][pl.ds(s, LANES)] = l00 + xa
                iref[1][pl.ds(s, LANES)] = l00 + xb
                iref[2][pl.ds(s, LANES)] = l01 + xa
                iref[3][pl.ds(s, LANES)] = l01 + xb
                iref[4][pl.ds(s, LANES)] = l10 + xa
                iref[5][pl.ds(s, LANES)] = l10 + xb
                iref[6][pl.ds(s, LANES)] = l11 + xa
                iref[7][pl.ds(s, LANES)] = l11 + xb
            scale = jnp.float32(0.5 * (N2 - 1))
            hi = jnp.float32(N2 - 1)
            tx = jnp.minimum(jnp.maximum((gxv + 1.0) * scale, 0.0), hi)
            ty = jnp.minimum(jnp.maximum((gyv + 1.0) * scale, 0.0), hi)
            tz = jnp.minimum(jnp.maximum((gzv + 1.0) * scale, 0.0), hi)
            x0 = tx.astype(jnp.int32)
            y0 = ty.astype(jnp.int32)
            z0 = tz.astype(jnp.int32)
            sl["w2"][0, pl.ds(s, LANES)] = tx - x0.astype(jnp.float32)
            sl["w2"][1, pl.ds(s, LANES)] = ty - y0.astype(jnp.float32)
            sl["w2"][2, pl.ds(s, LANES)] = tz - z0.astype(jnp.float32)
            x1 = jnp.minimum(x0 + 1, N2 - 1)
            y1 = jnp.minimum(y0 + 1, N2 - 1)
            z1 = jnp.minimum(z0 + 1, N2 - 1)
            sl["o2"][0][pl.ds(s, LANES)] = x0 & 7
            sl["o2"][1][pl.ds(s, LANES)] = x1 & 7
            xa = lax.shift_right_logical(x0, 3)
            xb = lax.shift_right_logical(x1, 3)
            l00 = (z0 * N2 + y0) * (N2 // 8)
            l01 = (z0 * N2 + y1) * (N2 // 8)
            l10 = (z1 * N2 + y0) * (N2 // 8)
            l11 = (z1 * N2 + y1) * (N2 // 8)
            rows = (l00 + xa, l00 + xb, l01 + xa, l01 + xb,
                    l10 + xa, l10 + xb, l11 + xa, l11 + xb)
            for j, r in enumerate(rows):
                sl["i2"][j][pl.ds(s, LANES)] = r
                sl["i2"][8 + j][pl.ds(s, LANES)] = r + V2ROWS
            return c2

        lax.fori_loop(0, NG, bloop, 0)

    def fire(sl):
        handles = []
        for v in range(2):
            for j in range(8):
                handles.append(
                    pltpu.async_copy(tabs[v].at[sl["i01"][v][j]],
                                     sl["b01"][v][j], sl["sem"]))
        for j in range(16):
            handles.append(pltpu.async_copy(t2.at[sl["i2"][j]],
                                            sl["b2"][j], sl["sem"]))
        return handles

    def interp_store(sl, pbase):
        def iloop(g, c2):
            s = g * LANES
            pv = lane + s

            def lerp3(vals, wx, wy, wz):
                c000, c001, c010, c011, c100, c101, c110, c111 = vals
                c00 = c000 + wx * (c001 - c000)
                c01 = c010 + wx * (c011 - c010)
                c10 = c100 + wx * (c101 - c100)
                c11 = c110 + wx * (c111 - c110)
                c0 = c00 + wy * (c01 - c00)
                c1 = c10 + wy * (c11 - c10)
                return c0 + wz * (c1 - c0)

            wrefs = (sl["w0"], sl["w1"])
            for v, (n, c, grp, sh, cb) in enumerate(VOLS):
                wx = wrefs[v][0, pl.ds(s, LANES)]
                wy = wrefs[v][1, pl.ds(s, LANES)]
                wz = wrefs[v][2, pl.ds(s, LANES)]
                oa = sl["o01"][v][0][pl.ds(s, LANES)]
                obv = sl["o01"][v][1][pl.ds(s, LANES)]
                bufs = sl["b01"][v]
                for ch in range(c):
                    cav = oa + ch
                    cbv = obv + ch
                    vals = tuple(
                        plsc.load_gather(bufs[j],
                                         [pv, cav if j % 2 == 0 else cbv])
                        for j in range(8))
                    ob[cb + ch, pl.ds(s, LANES)] = lerp3(vals, wx, wy, wz)
            wx = sl["w2"][0, pl.ds(s, LANES)]
            wy = sl["w2"][1, pl.ds(s, LANES)]
            wz = sl["w2"][2, pl.ds(s, LANES)]
            oav = sl["o2"][0][pl.ds(s, LANES)]
            obv = sl["o2"][1][pl.ds(s, LANES)]
            for ch in range(2):
                vals = tuple(
                    plsc.load_gather(sl["b2"][ch * 8 + j],
                                     [pv, oav if j % 2 == 0 else obv])
                    for j in range(8))
                ob[12 + ch, pl.ds(s, LANES)] = lerp3(vals, wx, wy, wz)
            return c2

        lax.fori_loop(0, NG, iloop, 0)
        pltpu.sync_copy(ob, out.at[:, pl.ds(pbase, CHUNK)])

    def pair_body(ci, carry):
        pa = wbase + ci * (2 * CHUNK)
        pb = pa + CHUNK
        build(slots[0], pa)
        ha = fire(slots[0])
        build(slots[1], pb)
        hb = fire(slots[1])
        for h in ha:
            h.wait()
        interp_store(slots[0], pa)
        for h in hb:
            h.wait()
        interp_store(slots[1], pb)
        return carry

    lax.fori_loop(0, NCHUNKS // 2, pair_body, 0)


_SLOT_SCRATCH = [
    pltpu.VMEM((3, CHUNK), jnp.float32),
    pltpu.VMEM((3, CHUNK), jnp.float32),
    pltpu.VMEM((3, CHUNK), jnp.float32),
    *[pltpu.VMEM((CHUNK,), jnp.int32) for _ in range(32)],
    *[pltpu.VMEM((CHUNK,), jnp.int32) for _ in range(6)],
    *[pltpu.VMEM((CHUNK, 8), jnp.float32) for _ in range(32)],
    pltpu.SemaphoreType.DMA,
]


@jax.jit
def kernel(grid, vol0, vol1, vol2):
    t2 = vol2.reshape(-1, 8)

    mesh = plsc.VectorSubcoreMesh(core_axis_name="c", subcore_axis_name="s")
    cp = pltpu.CompilerParams(
        needs_layout_passes=False, use_tc_tiling_on_sc=False)
    fmt = functools.partial(
        pl.kernel,
        mesh=mesh,
        out_type=(jax.ShapeDtypeStruct((V0 * 8,), jnp.float32),
                  jax.ShapeDtypeStruct((V1 * 4,), jnp.float32),
                  jax.ShapeDtypeStruct((P,), jnp.float32),
                  jax.ShapeDtypeStruct((P,), jnp.float32),
                  jax.ShapeDtypeStruct((P,), jnp.float32)),
        scratch_types=[
            pltpu.VMEM((8, K0), jnp.float32),
            pltpu.VMEM((4, K1), jnp.float32),
            pltpu.VMEM((8192,), jnp.float32),
            pltpu.VMEM((KG * 3,), jnp.float32),
            pltpu.VMEM((KG,), jnp.float32),
            pltpu.SemaphoreType.DMA,
        ],
        compiler_params=cp,
    )(_fmt_body)
    t0f, t1f, gx, gy, gz = fmt(vol0.reshape(8, V0), vol1.reshape(4, V1),
                               grid.reshape(P * 3))
    t0 = t0f.reshape(-1, 8)
    t1 = t1f.reshape(-1, 8)

    run = functools.partial(
        pl.kernel,
        mesh=mesh,
        out_type=jax.ShapeDtypeStruct((CTOT, P), jnp.float32),
        scratch_types=[
            pltpu.VMEM((CHUNK,), jnp.float32),
            pltpu.VMEM((CHUNK,), jnp.float32),
            pltpu.VMEM((CHUNK,), jnp.float32),
            pltpu.VMEM((CTOT, CHUNK), jnp.float32),
            *_SLOT_SCRATCH,
            *_SLOT_SCRATCH,
        ],
        compiler_params=cp,
    )(_sc_body)
    out = run(t0, t1, t2, gx, gy, gz)
    return out.reshape(1, CTOT, 1, 1, P)


# speedup vs baseline: 1.0450x; 1.0450x over previous
import functools

import jax
import jax.numpy as jnp
from jax import lax
from jax.experimental import pallas as pl
from jax.experimental.pallas import tpu as pltpu
from jax.experimental.pallas import tpu_sc as plsc

P = 524288
NC = 2
NS = 16
NW = NC * NS
PPW = P // NW
CHUNK = 128
NCHUNKS = PPW // CHUNK
LANES = 16
NG = CHUNK // LANES

VOLS = ((64, 8, 1, 0, 0), (128, 4, 2, 1, 8))
CTOT = 14
RPL = 64
N2 = 256
V2ROWS = N2 * N2 * (N2 // 8)

V0 = 64 * 64 * 64
V1 = 128 * 128 * 128
K0 = 1024
K1 = 2048
KG = 2048


def _fmt_body(v0, v1, t0f, t1f, inb0, inb1, outb, sem):
    wid = lax.axis_index("s") * NC + lax.axis_index("c")
    lane = lax.iota(jnp.int32, LANES)

    def do_vol(vin, tout, inb, c, k, nchunks, wvox, sh):
        cvec = lane & (c - 1)
        vbase = lane >> sh
        step = 16 // c

        def chunk(ci, carry):
            base = wid * wvox + ci * k
            handles = [pltpu.async_copy(vin.at[ch, pl.ds(base, k)],
                                        inb.at[ch], sem) for ch in range(c)]
            for h in handles:
                h.wait()

            def vloop(m, c2):
                for u in range(8):
                    mm = m * 8 + u
                    vox = vbase + mm * step
                    outb[pl.ds(mm * LANES, LANES)] = plsc.load_gather(
                        inb, [cvec, vox])
                return c2

            lax.fori_loop(0, k * c // (LANES * 8), vloop, 0)
            pltpu.sync_copy(outb.at[pl.ds(0, k * c)],
                            tout.at[pl.ds(base * c, k * c)])
            return carry

        lax.fori_loop(0, nchunks, chunk, 0)

    do_vol(v0, t0f, inb0, 8, K0, V0 // NW // K0, V0 // NW, 3)
    do_vol(v1, t1f, inb1, 4, K1, V1 // NW // K1, V1 // NW, 2)


def _slot(refs, base):
    return dict(
        w0=refs[base + 0], w1=refs[base + 1], w2=refs[base + 2],
        i01=(refs[base + 3:base + 11], refs[base + 11:base + 19]),
        i2=refs[base + 19:base + 35],
        o01=(refs[base + 35:base + 37], refs[base + 37:base + 39]),
        o2=(refs[base + 39], refs[base + 40]),
        b01=(refs[base + 41:base + 49], refs[base + 49:base + 57]),
        b2=refs[base + 57:base + 73],
        sem=refs[base + 73],
    )


def _sc_body(t0, t1, t2, g2, out, *refs):
    cg, ob = refs[:2]
    slots = (_slot(refs, 2), _slot(refs, 2 + 74))
    wid = lax.axis_index("s") * NC + lax.axis_index("c")
    wbase = wid * PPW
    tabs = (t0, t1)
    lane = lax.iota(jnp.int32, LANES)

    zero = jnp.zeros((LANES,), jnp.int32)

    def build(sl, pbase):
        pltpu.sync_copy(g2.at[pl.ds(pbase, CHUNK)], cg)
        wrefs = (sl["w0"], sl["w1"])

        def bloop(g, c2):
            s = g * LANES
            pv0 = lane + s
            gxv = plsc.load_gather(cg, [pv0, zero])
            gyv = plsc.load_gather(cg, [pv0, zero + 1])
            gzv = plsc.load_gather(cg, [pv0, zero + 2])
            for v, (n, c, grp, sh, _cb) in enumerate(VOLS):
                scale = jnp.float32(0.5 * (n - 1))
                hi = jnp.float32(n - 1)
                tx = jnp.minimum(jnp.maximum((gxv + 1.0) * scale, 0.0), hi)
                ty = jnp.minimum(jnp.maximum((gyv + 1.0) * scale, 0.0), hi)
                tz = jnp.minimum(jnp.maximum((gzv + 1.0) * scale, 0.0), hi)
                x0 = tx.astype(jnp.int32)
                y0 = ty.astype(jnp.int32)
                z0 = tz.astype(jnp.int32)
                wrefs[v][0, pl.ds(s, LANES)] = tx - x0.astype(jnp.float32)
                wrefs[v][1, pl.ds(s, LANES)] = ty - y0.astype(jnp.float32)
                wrefs[v][2, pl.ds(s, LANES)] = tz - z0.astype(jnp.float32)
                x1 = jnp.minimum(x0 + 1, n - 1)
                y1 = jnp.minimum(y0 + 1, n - 1)
                z1 = jnp.minimum(z0 + 1, n - 1)
                xa = lax.shift_right_logical(x0, sh)
                xb = lax.shift_right_logical(x1, sh)
                sl["o01"][v][0][pl.ds(s, LANES)] = (x0 & (grp - 1)) * c
                sl["o01"][v][1][pl.ds(s, LANES)] = (x1 & (grp - 1)) * c
                l00 = (z0 * n + y0) * RPL
                l01 = (z0 * n + y1) * RPL
                l10 = (z1 * n + y0) * RPL
                l11 = (z1 * n + y1) * RPL
                iref = sl["i01"][v]
                iref[0][pl.ds(s, LANES)] = l00 + xa
                iref[1][pl.ds(s, LANES)] = l00 + xb
                iref[2][pl.ds(s, LANES)] = l01 + xa
                iref[3][pl.ds(s, LANES)] = l01 + xb
                iref[4][pl.ds(s, LANES)] = l10 + xa
                iref[5][pl.ds(s, LANES)] = l10 + xb
                iref[6][pl.ds(s, LANES)] = l11 + xa
                iref[7][pl.ds(s, LANES)] = l11 + xb
            scale = jnp.float32(0.5 * (N2 - 1))
            hi = jnp.float32(N2 - 1)
            tx = jnp.minimum(jnp.maximum((gxv + 1.0) * scale, 0.0), hi)
            ty = jnp.minimum(jnp.maximum((gyv + 1.0) * scale, 0.0), hi)
            tz = jnp.minimum(jnp.maximum((gzv + 1.0) * scale, 0.0), hi)
            x0 = tx.astype(jnp.int32)
            y0 = ty.astype(jnp.int32)
            z0 = tz.astype(jnp.int32)
            sl["w2"][0, pl.ds(s, LANES)] = tx - x0.astype(jnp.float32)
            sl["w2"][1, pl.ds(s, LANES)] = ty - y0.astype(jnp.float32)
            sl["w2"][2, pl.ds(s, LANES)] = tz - z0.astype(jnp.float32)
            x1 = jnp.minimum(x0 + 1, N2 - 1)
            y1 = jnp.minimum(y0 + 1, N2 - 1)
            z1 = jnp.minimum(z0 + 1, N2 - 1)
            sl["o2"][0][pl.ds(s, LANES)] = x0 & 7
            sl["o2"][1][pl.ds(s, LANES)] = x1 & 7
            xa = lax.shift_right_logical(x0, 3)
            xb = lax.shift_right_logical(x1, 3)
            l00 = (z0 * N2 + y0) * (N2 // 8)
            l01 = (z0 * N2 + y1) * (N2 // 8)
            l10 = (z1 * N2 + y0) * (N2 // 8)
            l11 = (z1 * N2 + y1) * (N2 // 8)
            rows = (l00 + xa, l00 + xb, l01 + xa, l01 + xb,
                    l10 + xa, l10 + xb, l11 + xa, l11 + xb)
            for j, r in enumerate(rows):
                sl["i2"][j][pl.ds(s, LANES)] = r
                sl["i2"][8 + j][pl.ds(s, LANES)] = r + V2ROWS
            return c2

        lax.fori_loop(0, NG, bloop, 0)

    def fire(sl):
        handles = []
        for v in range(2):
            for j in range(8):
                handles.append(
                    pltpu.async_copy(tabs[v].at[sl["i01"][v][j]],
                                     sl["b01"][v][j], sl["sem"]))
        for j in range(16):
            handles.append(pltpu.async_copy(t2.at[sl["i2"][j]],
                                            sl["b2"][j], sl["sem"]))
        return handles

    def interp_store(sl, pbase):
        def iloop(g, c2):
            s = g * LANES
            pv = lane + s

            def lerp3(vals, wx, wy, wz):
                c000, c001, c010, c011, c100, c101, c110, c111 = vals
                c00 = c000 + wx * (c001 - c000)
                c01 = c010 + wx * (c011 - c010)
                c10 = c100 + wx * (c101 - c100)
                c11 = c110 + wx * (c111 - c110)
                c0 = c00 + wy * (c01 - c00)
                c1 = c10 + wy * (c11 - c10)
                return c0 + wz * (c1 - c0)

            wrefs = (sl["w0"], sl["w1"])
            for v, (n, c, grp, sh, cb) in enumerate(VOLS):
                wx = wrefs[v][0, pl.ds(s, LANES)]
                wy = wrefs[v][1, pl.ds(s, LANES)]
                wz = wrefs[v][2, pl.ds(s, LANES)]
                oa = sl["o01"][v][0][pl.ds(s, LANES)]
                obv = sl["o01"][v][1][pl.ds(s, LANES)]
                bufs = sl["b01"][v]
                for ch in range(c):
                    cav = oa + ch
                    cbv = obv + ch
                    vals = tuple(
                        plsc.load_gather(bufs[j],
                                         [pv, cav if j % 2 == 0 else cbv])
                        for j in range(8))
                    ob[cb + ch, pl.ds(s, LANES)] = lerp3(vals, wx, wy, wz)
            wx = sl["w2"][0, pl.ds(s, LANES)]
            wy = sl["w2"][1, pl.ds(s, LANES)]
            wz = sl["w2"][2, pl.ds(s, LANES)]
            oav = sl["o2"][0][pl.ds(s, LANES)]
            obv = sl["o2"][1][pl.ds(s, LANES)]
            for ch in range(2):
                vals = tuple(
                    plsc.load_gather(sl["b2"][ch * 8 + j],
                                     [pv, oav if j % 2 == 0 else obv])
                    for j in range(8))
                ob[12 + ch, pl.ds(s, LANES)] = lerp3(vals, wx, wy, wz)
            return c2

        lax.fori_loop(0, NG, iloop, 0)
        pltpu.sync_copy(ob, out.at[:, pl.ds(pbase, CHUNK)])

    def pair_body(ci, carry):
        pa = wbase + ci * (2 * CHUNK)
        pb = pa + CHUNK
        build(slots[0], pa)
        ha = fire(slots[0])
        build(slots[1], pb)
        hb = fire(slots[1])
        for h in ha:
            h.wait()
        interp_store(slots[0], pa)
        for h in hb:
            h.wait()
        interp_store(slots[1], pb)
        return carry

    lax.fori_loop(0, NCHUNKS // 2, pair_body, 0)


_SLOT_SCRATCH = [
    pltpu.VMEM((3, CHUNK), jnp.float32),
    pltpu.VMEM((3, CHUNK), jnp.float32),
    pltpu.VMEM((3, CHUNK), jnp.float32),
    *[pltpu.VMEM((CHUNK,), jnp.int32) for _ in range(32)],
    *[pltpu.VMEM((CHUNK,), jnp.int32) for _ in range(6)],
    *[pltpu.VMEM((CHUNK, 8), jnp.float32) for _ in range(32)],
    pltpu.SemaphoreType.DMA,
]


@jax.jit
def kernel(grid, vol0, vol1, vol2):
    t2 = vol2.reshape(-1, 8)

    mesh = plsc.VectorSubcoreMesh(core_axis_name="c", subcore_axis_name="s")
    cp = pltpu.CompilerParams(
        needs_layout_passes=False, use_tc_tiling_on_sc=False)
    fmt = functools.partial(
        pl.kernel,
        mesh=mesh,
        out_type=(jax.ShapeDtypeStruct((V0 * 8,), jnp.float32),
                  jax.ShapeDtypeStruct((V1 * 4,), jnp.float32)),
        scratch_types=[
            pltpu.VMEM((8, K0), jnp.float32),
            pltpu.VMEM((4, K1), jnp.float32),
            pltpu.VMEM((8192,), jnp.float32),
            pltpu.SemaphoreType.DMA,
        ],
        compiler_params=cp,
    )(_fmt_body)
    t0f, t1f = fmt(vol0.reshape(8, V0), vol1.reshape(4, V1))
    t0 = t0f.reshape(-1, 8)
    t1 = t1f.reshape(-1, 8)

    run = functools.partial(
        pl.kernel,
        mesh=mesh,
        out_type=jax.ShapeDtypeStruct((CTOT, P), jnp.float32),
        scratch_types=[
            pltpu.VMEM((CHUNK, 3), jnp.float32),
            pltpu.VMEM((CTOT, CHUNK), jnp.float32),
            *_SLOT_SCRATCH,
            *_SLOT_SCRATCH,
        ],
        compiler_params=cp,
    )(_sc_body)
    out = run(t0, t1, t2, grid.reshape(P, 3))
    return out.reshape(1, CTOT, 1, 1, P)


# speedup vs baseline: 1.5113x; 1.4462x over previous
import functools

import jax
import jax.numpy as jnp
from jax import lax
from jax.experimental import pallas as pl
from jax.experimental.pallas import tpu as pltpu
from jax.experimental.pallas import tpu_sc as plsc

P = 524288
NC = 2
NS = 16
NW = NC * NS
PPW = P // NW
CHUNK = 128
NCHUNKS = PPW // CHUNK
LANES = 16
NG = CHUNK // LANES

VOLS = ((64, 8, 1, 0, 0), (128, 4, 2, 1, 8))
CTOT = 14
RPL = 64
N2 = 256
V2ROWS = N2 * N2 * (N2 // 8)

V0 = 64 * 64 * 64
V1 = 128 * 128 * 128
K0 = 1024
K1 = 2048
KG = 2048


def _fmt_body(v0, v1, t0f, t1f, inb0, inb1, outb, sem):
    wid = lax.axis_index("s") * NC + lax.axis_index("c")
    lane = lax.iota(jnp.int32, LANES)

    def do_vol(vin, tout, inb, c, k, nchunks, wvox, sh):
        cvec = lane & (c - 1)
        vbase = lane >> sh
        step = 16 // c

        def chunk(ci, carry):
            base = wid * wvox + ci * k
            handles = [pltpu.async_copy(vin.at[ch, pl.ds(base, k)],
                                        inb.at[ch], sem) for ch in range(c)]
            for h in handles:
                h.wait()

            def vloop(m, c2):
                for u in range(8):
                    mm = m * 8 + u
                    vox = vbase + mm * step
                    outb[pl.ds(mm * LANES, LANES)] = plsc.load_gather(
                        inb, [cvec, vox])
                return c2

            lax.fori_loop(0, k * c // (LANES * 8), vloop, 0)
            pltpu.sync_copy(outb.at[pl.ds(0, k * c)],
                            tout.at[pl.ds(base * c, k * c)])
            return carry

        lax.fori_loop(0, nchunks, chunk, 0)

    do_vol(v0, t0f, inb0, 8, K0, V0 // NW // K0, V0 // NW, 3)
    do_vol(v1, t1f, inb1, 4, K1, V1 // NW // K1, V1 // NW, 2)


def _slot(refs, base):
    return dict(
        w0=refs[base + 0], w1=refs[base + 1], w2=refs[base + 2],
        i01=(refs[base + 3:base + 11], refs[base + 11:base + 19]),
        i2=refs[base + 19:base + 35],
        o01=(refs[base + 35:base + 37], refs[base + 37:base + 39]),
        o2=(refs[base + 39], refs[base + 40]),
        b01=(refs[base + 41:base + 49], refs[base + 49:base + 57]),
        b2=refs[base + 57:base + 73],
        sem=refs[base + 73],
    )


def _sc_body(t0, t1, t2, gx, gy, gz, out, *refs):
    cx, cy, cz, ob = refs[:4]
    slots = (_slot(refs, 4), _slot(refs, 4 + 74))
    wid = lax.axis_index("s") * NC + lax.axis_index("c")
    wbase = wid * PPW
    tabs = (t0, t1)
    lane = lax.iota(jnp.int32, LANES)

    def build(sl, pbase):
        pltpu.sync_copy(gx.at[pl.ds(pbase, CHUNK)], cx)
        pltpu.sync_copy(gy.at[pl.ds(pbase, CHUNK)], cy)
        pltpu.sync_copy(gz.at[pl.ds(pbase, CHUNK)], cz)
        wrefs = (sl["w0"], sl["w1"])

        def bloop(g, c2):
            s = g * LANES
            gxv = cx[pl.ds(s, LANES)]
            gyv = cy[pl.ds(s, LANES)]
            gzv = cz[pl.ds(s, LANES)]
            for v, (n, c, grp, sh, _cb) in enumerate(VOLS):
                scale = jnp.float32(0.5 * (n - 1))
                hi = jnp.float32(n - 1)
                tx = jnp.minimum(jnp.maximum((gxv + 1.0) * scale, 0.0), hi)
                ty = jnp.minimum(jnp.maximum((gyv + 1.0) * scale, 0.0), hi)
                tz = jnp.minimum(jnp.maximum((gzv + 1.0) * scale, 0.0), hi)
                x0 = tx.astype(jnp.int32)
                y0 = ty.astype(jnp.int32)
                z0 = tz.astype(jnp.int32)
                wrefs[v][0, pl.ds(s, LANES)] = tx - x0.astype(jnp.float32)
                wrefs[v][1, pl.ds(s, LANES)] = ty - y0.astype(jnp.float32)
                wrefs[v][2, pl.ds(s, LANES)] = tz - z0.astype(jnp.float32)
                x1 = jnp.minimum(x0 + 1, n - 1)
                y1 = jnp.minimum(y0 + 1, n - 1)
                z1 = jnp.minimum(z0 + 1, n - 1)
                xa = lax.shift_right_logical(x0, sh)
                xb = lax.shift_right_logical(x1, sh)
                sl["o01"][v][0][pl.ds(s, LANES)] = (x0 & (grp - 1)) * c
                sl["o01"][v][1][pl.ds(s, LANES)] = (x1 & (grp - 1)) * c
                l00 = (z0 * n + y0) * RPL
                l01 = (z0 * n + y1) * RPL
                l10 = (z1 * n + y0) * RPL
                l11 = (z1 * n + y1) * RPL
                iref = sl["i01"][v]
                iref[0][pl.ds(s, LANES)] = l00 + xa
                iref[1][pl.ds(s, LANES)] = l00 + xb
                iref[2][pl.ds(s, LANES)] = l01 + xa
                iref[3][pl.ds(s, LANES)] = l01 + xb
                iref[4][pl.ds(s, LANES)] = l10 + xa
                iref[5][pl.ds(s, LANES)] = l10 + xb
                iref[6][pl.ds(s, LANES)] = l11 + xa
                iref[7][pl.ds(s, LANES)] = l11 + xb
            scale = jnp.float32(0.5 * (N2 - 1))
            hi = jnp.float32(N2 - 1)
            tx = jnp.minimum(jnp.maximum((gxv + 1.0) * scale, 0.0), hi)
            ty = jnp.minimum(jnp.maximum((gyv + 1.0) * scale, 0.0), hi)
            tz = jnp.minimum(jnp.maximum((gzv + 1.0) * scale, 0.0), hi)
            x0 = tx.astype(jnp.int32)
            y0 = ty.astype(jnp.int32)
            z0 = tz.astype(jnp.int32)
            sl["w2"][0, pl.ds(s, LANES)] = tx - x0.astype(jnp.float32)
            sl["w2"][1, pl.ds(s, LANES)] = ty - y0.astype(jnp.float32)
            sl["w2"][2, pl.ds(s, LANES)] = tz - z0.astype(jnp.float32)
            x1 = jnp.minimum(x0 + 1, N2 - 1)
            y1 = jnp.minimum(y0 + 1, N2 - 1)
            z1 = jnp.minimum(z0 + 1, N2 - 1)
            sl["o2"][0][pl.ds(s, LANES)] = x0 & 7
            sl["o2"][1][pl.ds(s, LANES)] = x1 & 7
            xa = lax.shift_right_logical(x0, 3)
            xb = lax.shift_right_logical(x1, 3)
            l00 = (z0 * N2 + y0) * (N2 // 8)
            l01 = (z0 * N2 + y1) * (N2 // 8)
            l10 = (z1 * N2 + y0) * (N2 // 8)
            l11 = (z1 * N2 + y1) * (N2 // 8)
            rows = (l00 + xa, l00 + xb, l01 + xa, l01 + xb,
                    l10 + xa, l10 + xb, l11 + xa, l11 + xb)
            for j, r in enumerate(rows):
                sl["i2"][j][pl.ds(s, LANES)] = r
                sl["i2"][8 + j][pl.ds(s, LANES)] = r + V2ROWS
            return c2

        lax.fori_loop(0, NG, bloop, 0)

    def fire(sl):
        handles = []
        for v in range(2):
            for j in range(8):
                handles.append(
                    pltpu.async_copy(tabs[v].at[sl["i01"][v][j]],
                                     sl["b01"][v][j], sl["sem"]))
        for j in range(16):
            handles.append(pltpu.async_copy(t2.at[sl["i2"][j]],
                                            sl["b2"][j], sl["sem"]))
        return handles

    def interp_store(sl, pbase):
        def iloop(g, c2):
            s = g * LANES
            pv = lane + s

            def lerp3(vals, wx, wy, wz):
                c000, c001, c010, c011, c100, c101, c110, c111 = vals
                c00 = c000 + wx * (c001 - c000)
                c01 = c010 + wx * (c011 - c010)
                c10 = c100 + wx * (c101 - c100)
                c11 = c110 + wx * (c111 - c110)
                c0 = c00 + wy * (c01 - c00)
                c1 = c10 + wy * (c11 - c10)
                return c0 + wz * (c1 - c0)

            wrefs = (sl["w0"], sl["w1"])
            for v, (n, c, grp, sh, cb) in enumerate(VOLS):
                wx = wrefs[v][0, pl.ds(s, LANES)]
                wy = wrefs[v][1, pl.ds(s, LANES)]
                wz = wrefs[v][2, pl.ds(s, LANES)]
                oa = sl["o01"][v][0][pl.ds(s, LANES)]
                obv = sl["o01"][v][1][pl.ds(s, LANES)]
                bufs = sl["b01"][v]
                for ch in range(c):
                    cav = oa + ch
                    cbv = obv + ch
                    vals = tuple(
                        plsc.load_gather(bufs[j],
                                         [pv, cav if j % 2 == 0 else cbv])
                        for j in range(8))
                    ob[cb + ch, pl.ds(s, LANES)] = lerp3(vals, wx, wy, wz)
            wx = sl["w2"][0, pl.ds(s, LANES)]
            wy = sl["w2"][1, pl.ds(s, LANES)]
            wz = sl["w2"][2, pl.ds(s, LANES)]
            oav = sl["o2"][0][pl.ds(s, LANES)]
            obv = sl["o2"][1][pl.ds(s, LANES)]
            for ch in range(2):
                vals = tuple(
                    plsc.load_gather(sl["b2"][ch * 8 + j],
                                     [pv, oav if j % 2 == 0 else obv])
                    for j in range(8))
                ob[12 + ch, pl.ds(s, LANES)] = lerp3(vals, wx, wy, wz)
            return c2

        lax.fori_loop(0, NG, iloop, 0)
        pltpu.sync_copy(ob, out.at[:, pl.ds(pbase, CHUNK)])

    def pair_body(ci, carry):
        pa = wbase + ci * (2 * CHUNK)
        pb = pa + CHUNK
        build(slots[0], pa)
        ha = fire(slots[0])
        build(slots[1], pb)
        hb = fire(slots[1])
        for h in ha:
            h.wait()
        interp_store(slots[0], pa)
        for h in hb:
            h.wait()
        interp_store(slots[1], pb)
        return carry

    lax.fori_loop(0, NCHUNKS // 2, pair_body, 0)


_SLOT_SCRATCH = [
    pltpu.VMEM((3, CHUNK), jnp.float32),
    pltpu.VMEM((3, CHUNK), jnp.float32),
    pltpu.VMEM((3, CHUNK), jnp.float32),
    *[pltpu.VMEM((CHUNK,), jnp.int32) for _ in range(32)],
    *[pltpu.VMEM((CHUNK,), jnp.int32) for _ in range(6)],
    *[pltpu.VMEM((CHUNK, 8), jnp.float32) for _ in range(32)],
    pltpu.SemaphoreType.DMA,
]


@jax.jit
def kernel(grid, vol0, vol1, vol2):
    t2 = vol2.reshape(-1, 8)

    mesh = plsc.VectorSubcoreMesh(core_axis_name="c", subcore_axis_name="s")
    cp = pltpu.CompilerParams(
        needs_layout_passes=False, use_tc_tiling_on_sc=False)
    fmt = functools.partial(
        pl.kernel,
        mesh=mesh,
        out_type=(jax.ShapeDtypeStruct((V0 * 8,), jnp.float32),
                  jax.ShapeDtypeStruct((V1 * 4,), jnp.float32)),
        scratch_types=[
            pltpu.VMEM((8, K0), jnp.float32),
            pltpu.VMEM((4, K1), jnp.float32),
            pltpu.VMEM((8192,), jnp.float32),
            pltpu.SemaphoreType.DMA,
        ],
        compiler_params=cp,
    )(_fmt_body)
    t0f, t1f = fmt(vol0.reshape(8, V0), vol1.reshape(4, V1))
    t0 = t0f.reshape(-1, 8)
    t1 = t1f.reshape(-1, 8)

    run = functools.partial(
        pl.kernel,
        mesh=mesh,
        out_type=jax.ShapeDtypeStruct((CTOT, P), jnp.float32),
        scratch_types=[
            pltpu.VMEM((CHUNK,), jnp.float32),
            pltpu.VMEM((CHUNK,), jnp.float32),
            pltpu.VMEM((CHUNK,), jnp.float32),
            pltpu.VMEM((CTOT, CHUNK), jnp.float32),
            *_SLOT_SCRATCH,
            *_SLOT_SCRATCH,
        ],
        compiler_params=cp,
    )(_sc_body)
    g = grid.reshape(P, 3)
    out = run(t0, t1, t2, g[:, 0], g[:, 1], g[:, 2])
    return out.reshape(1, CTOT, 1, 1, P)
